# Initial kernel scaffold; baseline (speedup 1.0000x reference)
import jax
import jax.numpy as jnp
from jax.experimental import pallas as pl


def kernel(type_idx, hex1, hex2, type_emb):
    raise NotImplementedError("write your pallas kernel here")


# speedup vs baseline: 1.7192x; 1.7192x over previous
import functools

import jax
import jax.numpy as jnp
from jax import lax
from jax.experimental import pallas as pl
from jax.experimental.pallas import tpu as pltpu
from jax.experimental.pallas import tpu_sc as plsc

WIDTH_FULL = 17
WIDTH_PLAYABLE = 15
HEIGHT = 11
EMB_DIM = 8
OUT_W = EMB_DIM + 6

_NC = 2
_NS = 16
_NW = _NC * _NS
_L = 16


def _hex_features(h):
    y = (h * 241) >> 12
    x = h - y * WIDTH_FULL
    xc = jnp.minimum(x, WIDTH_PLAYABLE - 1)
    yc = jnp.minimum(y, HEIGHT - 1)
    vf = jnp.where(h >= 0, 1.0, 0.0).astype(jnp.float32)
    fx = xc.astype(jnp.float32) * (1.0 / (WIDTH_PLAYABLE - 1))
    fy = yc.astype(jnp.float32) * (1.0 / (HEIGHT - 1))
    return fx * vf, fy * vf, vf


def _make_kernel(batch):
    b_per_w = batch // _NW
    n_chunks = b_per_w // _L
    out_per_w = b_per_w * OUT_W
    mesh = plsc.VectorSubcoreMesh(core_axis_name="c", subcore_axis_name="s")

    @functools.partial(
        pl.kernel,
        mesh=mesh,
        out_type=jax.ShapeDtypeStruct((batch * OUT_W,), jnp.float32),
        compiler_params=pltpu.CompilerParams(needs_layout_passes=False),
        scratch_types=[
            pltpu.VMEM((b_per_w,), jnp.int32),
            pltpu.VMEM((b_per_w,), jnp.int32),
            pltpu.VMEM((b_per_w,), jnp.int32),
            pltpu.VMEM((4 * EMB_DIM,), jnp.float32),
            pltpu.VMEM((out_per_w,), jnp.float32),
        ],
    )
    def k(t_hbm, h1_hbm, h2_hbm, tab_hbm, out_hbm, t_v, h1_v, h2_v, tab_v, out_v):
        wid = lax.axis_index("s") * _NC + lax.axis_index("c")
        base = wid * b_per_w
        pltpu.sync_copy(tab_hbm, tab_v)
        pltpu.sync_copy(t_hbm.at[pl.ds(base, b_per_w)], t_v)
        pltpu.sync_copy(h1_hbm.at[pl.ds(base, b_per_w)], h1_v)
        pltpu.sync_copy(h2_hbm.at[pl.ds(base, b_per_w)], h2_v)

        iota14 = lax.iota(jnp.int32, _L) * OUT_W
        for c in range(n_chunks):
            sl = pl.ds(c * _L, _L)
            t = t_v[sl]
            ei = t * EMB_DIM
            idxv = iota14 + (c * _L * OUT_W)
            for j in range(EMB_DIM):
                e = plsc.load_gather(tab_v, [ei + j])
                plsc.store_scatter(out_v, [idxv + j], e)
            fx1, fy1, v1 = _hex_features(h1_v[sl])
            fx2, fy2, v2 = _hex_features(h2_v[sl])
            plsc.store_scatter(out_v, [idxv + EMB_DIM], fx1)
            plsc.store_scatter(out_v, [idxv + (EMB_DIM + 1)], fy1)
            plsc.store_scatter(out_v, [idxv + (EMB_DIM + 2)], v1)
            plsc.store_scatter(out_v, [idxv + (EMB_DIM + 3)], fx2)
            plsc.store_scatter(out_v, [idxv + (EMB_DIM + 4)], fy2)
            plsc.store_scatter(out_v, [idxv + (EMB_DIM + 5)], v2)

        pltpu.sync_copy(out_v, out_hbm.at[pl.ds(base * OUT_W, out_per_w)])

    return k


def kernel(type_idx, hex1, hex2, type_emb):
    batch = type_idx.shape[0]
    k = _make_kernel(batch)
    out_flat = k(
        type_idx.astype(jnp.int32),
        hex1.astype(jnp.int32),
        hex2.astype(jnp.int32),
        type_emb.reshape(-1).astype(jnp.float32),
    )
    return out_flat.reshape(batch, OUT_W)


# speedup vs baseline: 1.7848x; 1.0381x over previous
import functools

import jax
import jax.numpy as jnp
from jax import lax
from jax.experimental import pallas as pl
from jax.experimental.pallas import tpu as pltpu
from jax.experimental.pallas import tpu_sc as plsc

WIDTH_FULL = 17
WIDTH_PLAYABLE = 15
HEIGHT = 11
EMB_DIM = 8
OUT_W = EMB_DIM + 6

_NC = 2
_NS = 16
_NW = _NC * _NS
_L = 16
_GROUPS = 4


def _hex_features(h):
    y = (h * 241) >> 12
    x = h - y * WIDTH_FULL
    xc = jnp.minimum(x, WIDTH_PLAYABLE - 1)
    yc = jnp.minimum(y, HEIGHT - 1)
    vf = jnp.where(h >= 0, 1.0, 0.0).astype(jnp.float32)
    fx = xc.astype(jnp.float32) * (1.0 / (WIDTH_PLAYABLE - 1))
    fy = yc.astype(jnp.float32) * (1.0 / (HEIGHT - 1))
    return fx * vf, fy * vf, vf


def _make_kernel(batch):
    b_per_w = batch // _NW
    g_rows = b_per_w // _GROUPS
    n_chunks_g = g_rows // _L
    mesh = plsc.VectorSubcoreMesh(core_axis_name="c", subcore_axis_name="s")

    @functools.partial(
        pl.kernel,
        mesh=mesh,
        out_type=jax.ShapeDtypeStruct((batch * OUT_W,), jnp.float32),
        compiler_params=pltpu.CompilerParams(needs_layout_passes=False),
        scratch_types=[
            pltpu.VMEM((b_per_w,), jnp.int32),
            pltpu.VMEM((b_per_w,), jnp.int32),
            pltpu.VMEM((b_per_w,), jnp.int32),
            pltpu.VMEM((4 * EMB_DIM,), jnp.float32),
            pltpu.VMEM((b_per_w * OUT_W,), jnp.float32),
            pltpu.SemaphoreType.DMA,
            pltpu.SemaphoreType.DMA,
            pltpu.SemaphoreType.DMA,
            pltpu.SemaphoreType.DMA,
            [pltpu.SemaphoreType.DMA] * _GROUPS,
        ],
    )
    def k(t_hbm, h1_hbm, h2_hbm, tab_hbm, out_hbm, t_v, h1_v, h2_v, tab_v,
          out_v, sem_t, sem_h1, sem_h2, sem_tab, sem_g):
        wid = lax.axis_index("s") * _NC + lax.axis_index("c")
        base = wid * b_per_w
        cp_t = pltpu.async_copy(t_hbm.at[pl.ds(base, b_per_w)], t_v, sem_t)
        cp_h1 = pltpu.async_copy(h1_hbm.at[pl.ds(base, b_per_w)], h1_v, sem_h1)
        cp_h2 = pltpu.async_copy(h2_hbm.at[pl.ds(base, b_per_w)], h2_v, sem_h2)
        cp_tab = pltpu.async_copy(tab_hbm, tab_v, sem_tab)
        cp_t.wait()
        cp_h1.wait()
        cp_h2.wait()
        cp_tab.wait()

        iota14 = lax.iota(jnp.int32, _L) * OUT_W
        out_cps = []
        for g in range(_GROUPS):
            for cg in range(n_chunks_g):
                c = g * n_chunks_g + cg
                sl = pl.ds(c * _L, _L)
                t = t_v[sl]
                ei = t * EMB_DIM
                idxv = iota14 + (c * _L * OUT_W)
                for j in range(EMB_DIM):
                    e = plsc.load_gather(tab_v, [ei + j])
                    plsc.store_scatter(out_v, [idxv + j], e)
                fx1, fy1, v1 = _hex_features(h1_v[sl])
                fx2, fy2, v2 = _hex_features(h2_v[sl])
                plsc.store_scatter(out_v, [idxv + EMB_DIM], fx1)
                plsc.store_scatter(out_v, [idxv + (EMB_DIM + 1)], fy1)
                plsc.store_scatter(out_v, [idxv + (EMB_DIM + 2)], v1)
                plsc.store_scatter(out_v, [idxv + (EMB_DIM + 3)], fx2)
                plsc.store_scatter(out_v, [idxv + (EMB_DIM + 4)], fy2)
                plsc.store_scatter(out_v, [idxv + (EMB_DIM + 5)], v2)
            g_off = g * g_rows * OUT_W
            out_cps.append(pltpu.async_copy(
                out_v.at[pl.ds(g_off, g_rows * OUT_W)],
                out_hbm.at[pl.ds(base * OUT_W + g_off, g_rows * OUT_W)],
                sem_g[g]))
        for cp in out_cps:
            cp.wait()

    return k


def kernel(type_idx, hex1, hex2, type_emb):
    batch = type_idx.shape[0]
    k = _make_kernel(batch)
    out_flat = k(
        type_idx.astype(jnp.int32),
        hex1.astype(jnp.int32),
        hex2.astype(jnp.int32),
        type_emb.reshape(-1).astype(jnp.float32),
    )
    return out_flat.reshape(batch, OUT_W)


# speedup vs baseline: 2.1971x; 1.2310x over previous
import functools

import jax
import jax.numpy as jnp
from jax import lax
from jax.experimental import pallas as pl
from jax.experimental.pallas import tpu as pltpu
from jax.experimental.pallas import tpu_sc as plsc

WIDTH_FULL = 17
WIDTH_PLAYABLE = 15
HEIGHT = 11
EMB_DIM = 8
OUT_W = EMB_DIM + 6

_NC = 2
_NS = 16
_NW = _NC * _NS
_L = 16
_GROUPS = 4


def _hex_features(h):
    y = (h * 241) >> 12
    x = h - y * WIDTH_FULL
    xc = jnp.minimum(x, WIDTH_PLAYABLE - 1)
    yc = jnp.minimum(y, HEIGHT - 1)
    vf = jnp.where(h >= 0, 1.0, 0.0).astype(jnp.float32)
    fx = xc.astype(jnp.float32) * (1.0 / (WIDTH_PLAYABLE - 1))
    fy = yc.astype(jnp.float32) * (1.0 / (HEIGHT - 1))
    return fx * vf, fy * vf, vf


def _make_kernel(batch):
    b_per_w = batch // _NW
    g_rows = b_per_w // _GROUPS
    n_chunks_g = g_rows // _L
    mesh = plsc.VectorSubcoreMesh(core_axis_name="c", subcore_axis_name="s")

    @functools.partial(
        pl.kernel,
        mesh=mesh,
        out_type=jax.ShapeDtypeStruct((batch, OUT_W), jnp.float32),
        compiler_params=pltpu.CompilerParams(needs_layout_passes=False),
        scratch_types=[
            pltpu.VMEM((b_per_w,), jnp.int32),
            pltpu.VMEM((b_per_w,), jnp.int32),
            pltpu.VMEM((b_per_w,), jnp.int32),
            pltpu.VMEM((4 * EMB_DIM,), jnp.float32),
            pltpu.VMEM((b_per_w, OUT_W), jnp.float32),
            pltpu.SemaphoreType.DMA,
            pltpu.SemaphoreType.DMA,
            pltpu.SemaphoreType.DMA,
            pltpu.SemaphoreType.DMA,
            [pltpu.SemaphoreType.DMA] * _GROUPS,
        ],
    )
    def k(t_hbm, h1_hbm, h2_hbm, tab_hbm, out_hbm, t_v, h1_v, h2_v, tab_v,
          out_v, sem_t, sem_h1, sem_h2, sem_tab, sem_g):
        wid = lax.axis_index("s") * _NC + lax.axis_index("c")
        base = wid * b_per_w
        cp_t = pltpu.async_copy(t_hbm.at[pl.ds(base, b_per_w)], t_v, sem_t)
        cp_h1 = pltpu.async_copy(h1_hbm.at[pl.ds(base, b_per_w)], h1_v, sem_h1)
        cp_h2 = pltpu.async_copy(h2_hbm.at[pl.ds(base, b_per_w)], h2_v, sem_h2)
        cp_tab = pltpu.async_copy(tab_hbm, tab_v, sem_tab)
        cp_t.wait()
        cp_h1.wait()
        cp_h2.wait()
        cp_tab.wait()

        iota = lax.iota(jnp.int32, _L)
        cols = [jnp.full((_L,), j, jnp.int32) for j in range(OUT_W)]
        out_cps = []
        for g in range(_GROUPS):
            for cg in range(n_chunks_g):
                c = g * n_chunks_g + cg
                sl = pl.ds(c * _L, _L)
                t = t_v[sl]
                ei = t * EMB_DIM
                rowv = iota + (c * _L)
                for j in range(EMB_DIM):
                    e = plsc.load_gather(tab_v, [ei + j])
                    plsc.store_scatter(out_v, [rowv, cols[j]], e)
                fx1, fy1, v1 = _hex_features(h1_v[sl])
                fx2, fy2, v2 = _hex_features(h2_v[sl])
                plsc.store_scatter(out_v, [rowv, cols[EMB_DIM]], fx1)
                plsc.store_scatter(out_v, [rowv, cols[EMB_DIM + 1]], fy1)
                plsc.store_scatter(out_v, [rowv, cols[EMB_DIM + 2]], v1)
                plsc.store_scatter(out_v, [rowv, cols[EMB_DIM + 3]], fx2)
                plsc.store_scatter(out_v, [rowv, cols[EMB_DIM + 4]], fy2)
                plsc.store_scatter(out_v, [rowv, cols[EMB_DIM + 5]], v2)
            g_off = g * g_rows
            out_cps.append(pltpu.async_copy(
                out_v.at[pl.ds(g_off, g_rows), :],
                out_hbm.at[pl.ds(base + g_off, g_rows), :],
                sem_g[g]))
        for cp in out_cps:
            cp.wait()

    return k


def kernel(type_idx, hex1, hex2, type_emb):
    batch = type_idx.shape[0]
    k = _make_kernel(batch)
    return k(
        type_idx.astype(jnp.int32),
        hex1.astype(jnp.int32),
        hex2.astype(jnp.int32),
        type_emb.reshape(-1).astype(jnp.float32),
    )


# speedup vs baseline: 2.9098x; 1.3243x over previous
import functools

import jax
import jax.numpy as jnp
from jax import lax
from jax.experimental import pallas as pl
from jax.experimental.pallas import tpu as pltpu
from jax.experimental.pallas import tpu_sc as plsc

WIDTH_FULL = 17
WIDTH_PLAYABLE = 15
HEIGHT = 11
NUM_TYPES = 4
EMB_DIM = 8
OUT_W = EMB_DIM + 6

_NC = 2
_NS = 16
_NW = _NC * _NS
_L = 16
_GROUPS = 4


def _hex_features(h):
    y = (h * 241) >> 12
    x = h - y * WIDTH_FULL
    xc = jnp.minimum(x, WIDTH_PLAYABLE - 1)
    yc = jnp.minimum(y, HEIGHT - 1)
    vf = jnp.where(h >= 0, 1.0, 0.0).astype(jnp.float32)
    fx = xc.astype(jnp.float32) * (1.0 / (WIDTH_PLAYABLE - 1))
    fy = yc.astype(jnp.float32) * (1.0 / (HEIGHT - 1))
    return fx * vf, fy * vf, vf


def _make_kernel(batch):
    b_per_w = batch // _NW
    g_cols = b_per_w // _GROUPS
    n_chunks_g = g_cols // _L
    mesh = plsc.VectorSubcoreMesh(core_axis_name="c", subcore_axis_name="s")

    @functools.partial(
        pl.kernel,
        mesh=mesh,
        out_type=jax.ShapeDtypeStruct((OUT_W, batch), jnp.float32),
        compiler_params=pltpu.CompilerParams(needs_layout_passes=False),
        scratch_types=[
            pltpu.VMEM((b_per_w,), jnp.int32),
            pltpu.VMEM((b_per_w,), jnp.int32),
            pltpu.VMEM((b_per_w,), jnp.int32),
            pltpu.VMEM((NUM_TYPES, EMB_DIM), jnp.float32),
            pltpu.VMEM((OUT_W, b_per_w), jnp.float32),
            pltpu.SemaphoreType.DMA,
            pltpu.SemaphoreType.DMA,
            pltpu.SemaphoreType.DMA,
            pltpu.SemaphoreType.DMA,
            [pltpu.SemaphoreType.DMA] * _GROUPS,
        ],
    )
    def k(t_hbm, h1_hbm, h2_hbm, tab_hbm, out_hbm, t_v, h1_v, h2_v, tab_v,
          out_v, sem_t, sem_h1, sem_h2, sem_tab, sem_g):
        wid = lax.axis_index("s") * _NC + lax.axis_index("c")
        base = wid * b_per_w
        cp_t = pltpu.async_copy(t_hbm.at[pl.ds(base, b_per_w)], t_v, sem_t)
        cp_h1 = pltpu.async_copy(h1_hbm.at[pl.ds(base, b_per_w)], h1_v, sem_h1)
        cp_h2 = pltpu.async_copy(h2_hbm.at[pl.ds(base, b_per_w)], h2_v, sem_h2)
        cp_tab = pltpu.async_copy(tab_hbm, tab_v, sem_tab)
        cp_t.wait()
        cp_h1.wait()
        cp_h2.wait()
        cp_tab.wait()

        cols = [jnp.full((_L,), j, jnp.int32) for j in range(EMB_DIM)]
        out_cps = []
        for g in range(_GROUPS):
            for cg in range(n_chunks_g):
                c = g * n_chunks_g + cg
                sl = pl.ds(c * _L, _L)
                t = t_v[sl]
                for j in range(EMB_DIM):
                    out_v[j, sl] = plsc.load_gather(tab_v, [t, cols[j]])
                fx1, fy1, v1 = _hex_features(h1_v[sl])
                fx2, fy2, v2 = _hex_features(h2_v[sl])
                out_v[EMB_DIM, sl] = fx1
                out_v[EMB_DIM + 1, sl] = fy1
                out_v[EMB_DIM + 2, sl] = v1
                out_v[EMB_DIM + 3, sl] = fx2
                out_v[EMB_DIM + 4, sl] = fy2
                out_v[EMB_DIM + 5, sl] = v2
            g_off = g * g_cols
            out_cps.append(pltpu.async_copy(
                out_v.at[:, pl.ds(g_off, g_cols)],
                out_hbm.at[:, pl.ds(base + g_off, g_cols)],
                sem_g[g]))
        for cp in out_cps:
            cp.wait()

    return k


def kernel(type_idx, hex1, hex2, type_emb):
    batch = type_idx.shape[0]
    k = _make_kernel(batch)
    out_t = k(
        type_idx.astype(jnp.int32),
        hex1.astype(jnp.int32),
        hex2.astype(jnp.int32),
        type_emb.astype(jnp.float32),
    )
    return out_t.T


# speedup vs baseline: 3.0597x; 1.0515x over previous
import functools

import jax
import jax.numpy as jnp
from jax import lax
from jax.experimental import pallas as pl
from jax.experimental.pallas import tpu as pltpu
from jax.experimental.pallas import tpu_sc as plsc

WIDTH_FULL = 17
WIDTH_PLAYABLE = 15
HEIGHT = 11
NUM_TYPES = 4
EMB_DIM = 8
OUT_W = EMB_DIM + 6

_NC = 2
_NS = 16
_NW = _NC * _NS
_L = 16
_GROUPS = 4


def _hex_features(h):
    y = (h * 241) >> 12
    x = h - y * WIDTH_FULL
    xc = jnp.minimum(x, WIDTH_PLAYABLE - 1)
    yc = jnp.minimum(y, HEIGHT - 1)
    vf = jnp.where(h >= 0, 1.0, 0.0).astype(jnp.float32)
    fx = xc.astype(jnp.float32) * (1.0 / (WIDTH_PLAYABLE - 1))
    fy = yc.astype(jnp.float32) * (1.0 / (HEIGHT - 1))
    return fx * vf, fy * vf, vf


def _make_kernel(batch):
    b_per_w = batch // _NW
    g_cols = b_per_w // _GROUPS
    n_chunks_g = g_cols // _L
    mesh = plsc.VectorSubcoreMesh(core_axis_name="c", subcore_axis_name="s")

    @functools.partial(
        pl.kernel,
        mesh=mesh,
        out_type=jax.ShapeDtypeStruct((OUT_W, batch), jnp.float32),
        compiler_params=pltpu.CompilerParams(needs_layout_passes=False),
        scratch_types=[
            pltpu.VMEM((b_per_w,), jnp.int32),
            pltpu.VMEM((b_per_w,), jnp.int32),
            pltpu.VMEM((b_per_w,), jnp.int32),
            pltpu.VMEM((NUM_TYPES, EMB_DIM), jnp.float32),
            pltpu.VMEM((OUT_W, b_per_w), jnp.float32),
            pltpu.SemaphoreType.DMA,
            pltpu.SemaphoreType.DMA,
            pltpu.SemaphoreType.DMA,
            pltpu.SemaphoreType.DMA,
            [pltpu.SemaphoreType.DMA] * _GROUPS,
        ],
    )
    def k(t_hbm, h1_hbm, h2_hbm, tab_hbm, out_hbm, t_v, h1_v, h2_v, tab_v,
          out_v, sem_t, sem_h1, sem_h2, sem_tab, sem_g):
        wid = lax.axis_index("s") * _NC + lax.axis_index("c")
        base = wid * b_per_w
        cp_t = pltpu.async_copy(t_hbm.at[pl.ds(base, b_per_w)], t_v, sem_t)
        cp_h1 = pltpu.async_copy(h1_hbm.at[pl.ds(base, b_per_w)], h1_v, sem_h1)
        cp_h2 = pltpu.async_copy(h2_hbm.at[pl.ds(base, b_per_w)], h2_v, sem_h2)
        cp_tab = pltpu.async_copy(tab_hbm, tab_v, sem_tab)
        cp_t.wait()
        cp_h1.wait()
        cp_h2.wait()
        cp_tab.wait()

        cols = [jnp.full((_L,), j, jnp.int32) for j in range(EMB_DIM)]
        out_cps = []
        for g in range(_GROUPS):
            @pl.loop(0, n_chunks_g, unroll=2)
            def _chunk(cg, g=g):
                sl = pl.ds((g * n_chunks_g + cg) * _L, _L)
                t = t_v[sl]
                for j in range(EMB_DIM):
                    out_v[j, sl] = plsc.load_gather(tab_v, [t, cols[j]])
                fx1, fy1, v1 = _hex_features(h1_v[sl])
                fx2, fy2, v2 = _hex_features(h2_v[sl])
                out_v[EMB_DIM, sl] = fx1
                out_v[EMB_DIM + 1, sl] = fy1
                out_v[EMB_DIM + 2, sl] = v1
                out_v[EMB_DIM + 3, sl] = fx2
                out_v[EMB_DIM + 4, sl] = fy2
                out_v[EMB_DIM + 5, sl] = v2
            g_off = g * g_cols
            out_cps.append(pltpu.async_copy(
                out_v.at[:, pl.ds(g_off, g_cols)],
                out_hbm.at[:, pl.ds(base + g_off, g_cols)],
                sem_g[g]))
        for cp in out_cps:
            cp.wait()

    return k


def kernel(type_idx, hex1, hex2, type_emb):
    batch = type_idx.shape[0]
    k = _make_kernel(batch)
    out_t = k(
        type_idx.astype(jnp.int32),
        hex1.astype(jnp.int32),
        hex2.astype(jnp.int32),
        type_emb.astype(jnp.float32),
    )
    return out_t.T


# speedup vs baseline: 3.0785x; 1.0061x over previous
import functools

import jax
import jax.numpy as jnp
from jax import lax
from jax.experimental import pallas as pl
from jax.experimental.pallas import tpu as pltpu
from jax.experimental.pallas import tpu_sc as plsc

WIDTH_FULL = 17
WIDTH_PLAYABLE = 15
HEIGHT = 11
NUM_TYPES = 4
EMB_DIM = 8
OUT_W = EMB_DIM + 6

_NC = 2
_NS = 16
_NW = _NC * _NS
_L = 16
_GROUPS = 2


def _hex_features(h):
    y = (h * 241) >> 12
    x = h - y * WIDTH_FULL
    xc = jnp.minimum(x, WIDTH_PLAYABLE - 1)
    yc = jnp.minimum(y, HEIGHT - 1)
    vf = jnp.where(h >= 0, 1.0, 0.0).astype(jnp.float32)
    fx = xc.astype(jnp.float32) * (1.0 / (WIDTH_PLAYABLE - 1))
    fy = yc.astype(jnp.float32) * (1.0 / (HEIGHT - 1))
    return fx * vf, fy * vf, vf


def _make_kernel(batch):
    b_per_w = batch // _NW
    g_cols = b_per_w // _GROUPS
    n_chunks_g = g_cols // _L
    mesh = plsc.VectorSubcoreMesh(core_axis_name="c", subcore_axis_name="s")

    @functools.partial(
        pl.kernel,
        mesh=mesh,
        out_type=jax.ShapeDtypeStruct((OUT_W, batch), jnp.float32),
        compiler_params=pltpu.CompilerParams(needs_layout_passes=False),
        scratch_types=[
            pltpu.VMEM((b_per_w,), jnp.int32),
            pltpu.VMEM((b_per_w,), jnp.int32),
            pltpu.VMEM((b_per_w,), jnp.int32),
            pltpu.VMEM((NUM_TYPES, EMB_DIM), jnp.float32),
            pltpu.VMEM((OUT_W, b_per_w), jnp.float32),
            pltpu.SemaphoreType.DMA,
            pltpu.SemaphoreType.DMA,
            pltpu.SemaphoreType.DMA,
            pltpu.SemaphoreType.DMA,
            [pltpu.SemaphoreType.DMA] * _GROUPS,
        ],
    )
    def k(t_hbm, h1_hbm, h2_hbm, tab_hbm, out_hbm, t_v, h1_v, h2_v, tab_v,
          out_v, sem_t, sem_h1, sem_h2, sem_tab, sem_g):
        wid = lax.axis_index("s") * _NC + lax.axis_index("c")
        base = wid * b_per_w
        cp_t = pltpu.async_copy(t_hbm.at[pl.ds(base, b_per_w)], t_v, sem_t)
        cp_h1 = pltpu.async_copy(h1_hbm.at[pl.ds(base, b_per_w)], h1_v, sem_h1)
        cp_h2 = pltpu.async_copy(h2_hbm.at[pl.ds(base, b_per_w)], h2_v, sem_h2)
        cp_tab = pltpu.async_copy(tab_hbm, tab_v, sem_tab)
        cp_t.wait()
        cp_h1.wait()
        cp_h2.wait()
        cp_tab.wait()

        cols = [jnp.full((_L,), j, jnp.int32) for j in range(EMB_DIM)]
        out_cps = []
        for g in range(_GROUPS):
            @pl.loop(0, n_chunks_g, unroll=1)
            def _chunk(cg, g=g):
                sl = pl.ds((g * n_chunks_g + cg) * _L, _L)
                t = t_v[sl]
                for j in range(EMB_DIM):
                    out_v[j, sl] = plsc.load_gather(tab_v, [t, cols[j]])
                fx1, fy1, v1 = _hex_features(h1_v[sl])
                fx2, fy2, v2 = _hex_features(h2_v[sl])
                out_v[EMB_DIM, sl] = fx1
                out_v[EMB_DIM + 1, sl] = fy1
                out_v[EMB_DIM + 2, sl] = v1
                out_v[EMB_DIM + 3, sl] = fx2
                out_v[EMB_DIM + 4, sl] = fy2
                out_v[EMB_DIM + 5, sl] = v2
            g_off = g * g_cols
            out_cps.append(pltpu.async_copy(
                out_v.at[:, pl.ds(g_off, g_cols)],
                out_hbm.at[:, pl.ds(base + g_off, g_cols)],
                sem_g[g]))
        for cp in out_cps:
            cp.wait()

    return k


def kernel(type_idx, hex1, hex2, type_emb):
    batch = type_idx.shape[0]
    k = _make_kernel(batch)
    out_t = k(
        type_idx.astype(jnp.int32),
        hex1.astype(jnp.int32),
        hex2.astype(jnp.int32),
        type_emb.astype(jnp.float32),
    )
    return out_t.T


# speedup vs baseline: 3.4938x; 1.1349x over previous
import functools

import jax
import jax.numpy as jnp
from jax import lax
from jax.experimental import pallas as pl
from jax.experimental.pallas import tpu as pltpu
from jax.experimental.pallas import tpu_sc as plsc

WIDTH_FULL = 17
WIDTH_PLAYABLE = 15
HEIGHT = 11
NUM_TYPES = 4
EMB_DIM = 8
OUT_W = EMB_DIM + 6

_NC = 2
_NS = 16
_NW = _NC * _NS
_L = 16
_GROUPS = 2


def _hex_features(h):
    y = (h * 241) >> 12
    x = h - y * WIDTH_FULL
    xc = jnp.minimum(x, WIDTH_PLAYABLE - 1)
    yc = jnp.minimum(y, HEIGHT - 1)
    vf = jnp.where(h >= 0, 1.0, 0.0).astype(jnp.float32)
    fx = xc.astype(jnp.float32) * (1.0 / (WIDTH_PLAYABLE - 1))
    fy = yc.astype(jnp.float32) * (1.0 / (HEIGHT - 1))
    return fx * vf, fy * vf, vf


def _make_kernel(batch):
    b_per_w = batch // _NW
    g_cols = b_per_w // _GROUPS
    n_chunks_g = g_cols // _L
    mesh = plsc.VectorSubcoreMesh(core_axis_name="c", subcore_axis_name="s")

    @functools.partial(
        pl.kernel,
        mesh=mesh,
        out_type=jax.ShapeDtypeStruct((OUT_W, batch), jnp.float32),
        compiler_params=pltpu.CompilerParams(needs_layout_passes=False),
        scratch_types=[
            pltpu.VMEM((b_per_w,), jnp.int32),
            pltpu.VMEM((b_per_w,), jnp.int32),
            pltpu.VMEM((b_per_w,), jnp.int32),
            pltpu.VMEM((2, _L), jnp.float32),
            pltpu.VMEM((OUT_W, b_per_w), jnp.float32),
            pltpu.SemaphoreType.DMA,
            pltpu.SemaphoreType.DMA,
            pltpu.SemaphoreType.DMA,
            pltpu.SemaphoreType.DMA,
            [pltpu.SemaphoreType.DMA] * _GROUPS,
        ],
    )
    def k(t_hbm, h1_hbm, h2_hbm, tab_hbm, out_hbm, t_v, h1_v, h2_v, tab_v,
          out_v, sem_t, sem_h1, sem_h2, sem_tab, sem_g):
        wid = lax.axis_index("s") * _NC + lax.axis_index("c")
        base = wid * b_per_w
        cp_t = pltpu.async_copy(t_hbm.at[pl.ds(base, b_per_w)], t_v, sem_t)
        cp_h1 = pltpu.async_copy(h1_hbm.at[pl.ds(base, b_per_w)], h1_v, sem_h1)
        cp_h2 = pltpu.async_copy(h2_hbm.at[pl.ds(base, b_per_w)], h2_v, sem_h2)
        cp_tab = pltpu.async_copy(tab_hbm, tab_v, sem_tab)
        cp_t.wait()
        cp_h1.wait()
        cp_h2.wait()
        cp_tab.wait()

        rows = [tab_v[0, :], tab_v[1, :]]
        tab_b = [[jnp.full((_L,), 0.0, jnp.float32)
                  + rows[(r * EMB_DIM + j) // _L][(r * EMB_DIM + j) % _L]
                  for j in range(EMB_DIM)] for r in range(NUM_TYPES)]
        out_cps = []
        for g in range(_GROUPS):
            @pl.loop(0, n_chunks_g, unroll=2)
            def _chunk(cg, g=g):
                sl = pl.ds((g * n_chunks_g + cg) * _L, _L)
                t = t_v[sl]
                m0 = (t & 1) == 1
                m1 = t >= 2
                for j in range(EMB_DIM):
                    lo = jnp.where(m0, tab_b[1][j], tab_b[0][j])
                    hi = jnp.where(m0, tab_b[3][j], tab_b[2][j])
                    out_v[j, sl] = jnp.where(m1, hi, lo)
                fx1, fy1, v1 = _hex_features(h1_v[sl])
                fx2, fy2, v2 = _hex_features(h2_v[sl])
                out_v[EMB_DIM, sl] = fx1
                out_v[EMB_DIM + 1, sl] = fy1
                out_v[EMB_DIM + 2, sl] = v1
                out_v[EMB_DIM + 3, sl] = fx2
                out_v[EMB_DIM + 4, sl] = fy2
                out_v[EMB_DIM + 5, sl] = v2
            g_off = g * g_cols
            out_cps.append(pltpu.async_copy(
                out_v.at[:, pl.ds(g_off, g_cols)],
                out_hbm.at[:, pl.ds(base + g_off, g_cols)],
                sem_g[g]))
        for cp in out_cps:
            cp.wait()

    return k


def kernel(type_idx, hex1, hex2, type_emb):
    batch = type_idx.shape[0]
    k = _make_kernel(batch)
    out_t = k(
        type_idx.astype(jnp.int32),
        hex1.astype(jnp.int32),
        hex2.astype(jnp.int32),
        type_emb.astype(jnp.float32).reshape(2, _L),
    )
    return out_t.T


# speedup vs baseline: 3.5223x; 1.0081x over previous
import functools

import jax
import jax.numpy as jnp
from jax import lax
from jax.experimental import pallas as pl
from jax.experimental.pallas import tpu as pltpu
from jax.experimental.pallas import tpu_sc as plsc

WIDTH_FULL = 17
WIDTH_PLAYABLE = 15
HEIGHT = 11
NUM_TYPES = 4
EMB_DIM = 8
OUT_W = EMB_DIM + 6

_NC = 2
_NS = 16
_NW = _NC * _NS
_L = 16
_GROUPS = 2


def _hex_features(h):
    y = (h * 241) >> 12
    x = h - y * WIDTH_FULL
    xc = jnp.minimum(x, WIDTH_PLAYABLE - 1)
    yc = jnp.minimum(y, HEIGHT - 1)
    vf = jnp.where(h >= 0, 1.0, 0.0).astype(jnp.float32)
    fx = xc.astype(jnp.float32) * (1.0 / (WIDTH_PLAYABLE - 1))
    fy = yc.astype(jnp.float32) * (1.0 / (HEIGHT - 1))
    return fx * vf, fy * vf, vf


def _make_kernel(batch):
    b_per_w = batch // _NW
    g_cols = b_per_w // _GROUPS
    n_chunks_g = g_cols // _L
    mesh = plsc.VectorSubcoreMesh(core_axis_name="c", subcore_axis_name="s")

    @functools.partial(
        pl.kernel,
        mesh=mesh,
        out_type=jax.ShapeDtypeStruct((OUT_W, batch), jnp.float32),
        compiler_params=pltpu.CompilerParams(needs_layout_passes=False),
        scratch_types=[
            pltpu.VMEM((b_per_w,), jnp.int32),
            pltpu.VMEM((b_per_w,), jnp.int32),
            pltpu.VMEM((b_per_w,), jnp.int32),
            pltpu.VMEM((2, _L), jnp.float32),
            pltpu.VMEM((OUT_W, b_per_w), jnp.float32),
            pltpu.SemaphoreType.DMA,
            pltpu.SemaphoreType.DMA,
            pltpu.SemaphoreType.DMA,
            pltpu.SemaphoreType.DMA,
            [pltpu.SemaphoreType.DMA] * _GROUPS,
        ],
    )
    def k(t_hbm, h1_hbm, h2_hbm, tab_hbm, out_hbm, t_v, h1_v, h2_v, tab_v,
          out_v, sem_t, sem_h1, sem_h2, sem_tab, sem_g):
        wid = lax.axis_index("s") * _NC + lax.axis_index("c")
        base = wid * b_per_w
        cp_t = pltpu.async_copy(t_hbm.at[pl.ds(base, b_per_w)], t_v, sem_t)
        cp_h1 = pltpu.async_copy(h1_hbm.at[pl.ds(base, b_per_w)], h1_v, sem_h1)
        cp_h2 = pltpu.async_copy(h2_hbm.at[pl.ds(base, b_per_w)], h2_v, sem_h2)
        cp_tabs = [
            pltpu.async_copy(
                tab_hbm.at[r, :],
                tab_v.at[r // 2, pl.ds((r % 2) * EMB_DIM, EMB_DIM)],
                sem_tab)
            for r in range(NUM_TYPES)
        ]
        cp_t.wait()
        cp_h1.wait()
        cp_h2.wait()
        for cp in cp_tabs:
            cp.wait()

        rows = [tab_v[0, :], tab_v[1, :]]
        tab_b = [[jnp.full((_L,), 0.0, jnp.float32)
                  + rows[(r * EMB_DIM + j) // _L][(r * EMB_DIM + j) % _L]
                  for j in range(EMB_DIM)] for r in range(NUM_TYPES)]
        out_cps = []
        for g in range(_GROUPS):
            @pl.loop(0, n_chunks_g, unroll=4)
            def _chunk(cg, g=g):
                sl = pl.ds((g * n_chunks_g + cg) * _L, _L)
                t = t_v[sl]
                m0 = (t & 1) == 1
                m1 = t >= 2
                for j in range(EMB_DIM):
                    lo = jnp.where(m0, tab_b[1][j], tab_b[0][j])
                    hi = jnp.where(m0, tab_b[3][j], tab_b[2][j])
                    out_v[j, sl] = jnp.where(m1, hi, lo)
                fx1, fy1, v1 = _hex_features(h1_v[sl])
                fx2, fy2, v2 = _hex_features(h2_v[sl])
                out_v[EMB_DIM, sl] = fx1
                out_v[EMB_DIM + 1, sl] = fy1
                out_v[EMB_DIM + 2, sl] = v1
                out_v[EMB_DIM + 3, sl] = fx2
                out_v[EMB_DIM + 4, sl] = fy2
                out_v[EMB_DIM + 5, sl] = v2
            g_off = g * g_cols
            out_cps.append(pltpu.async_copy(
                out_v.at[:, pl.ds(g_off, g_cols)],
                out_hbm.at[:, pl.ds(base + g_off, g_cols)],
                sem_g[g]))
        for cp in out_cps:
            cp.wait()

    return k


def kernel(type_idx, hex1, hex2, type_emb):
    batch = type_idx.shape[0]
    k = _make_kernel(batch)
    out_t = k(
        type_idx.astype(jnp.int32),
        hex1.astype(jnp.int32),
        hex2.astype(jnp.int32),
        type_emb.astype(jnp.float32),
    )
    return out_t.T


# speedup vs baseline: 3.6438x; 1.0345x over previous
import functools

import jax
import jax.numpy as jnp
from jax import lax
from jax.experimental import pallas as pl
from jax.experimental.pallas import tpu as pltpu
from jax.experimental.pallas import tpu_sc as plsc

WIDTH_FULL = 17
WIDTH_PLAYABLE = 15
HEIGHT = 11
NUM_TYPES = 4
EMB_DIM = 8
OUT_W = EMB_DIM + 6

_NC = 2
_NS = 16
_NW = _NC * _NS
_L = 16
_GROUPS = 1


def _hex_features(h):
    y = (h * 241) >> 12
    x = h - y * WIDTH_FULL
    xc = jnp.minimum(x, WIDTH_PLAYABLE - 1)
    yc = jnp.minimum(y, HEIGHT - 1)
    vf = jnp.where(h >= 0, 1.0, 0.0).astype(jnp.float32)
    fx = xc.astype(jnp.float32) * (1.0 / (WIDTH_PLAYABLE - 1))
    fy = yc.astype(jnp.float32) * (1.0 / (HEIGHT - 1))
    return fx * vf, fy * vf, vf


def _make_kernel(batch):
    b_per_w = batch // _NW
    g_cols = b_per_w // _GROUPS
    n_chunks_g = g_cols // _L
    mesh = plsc.VectorSubcoreMesh(core_axis_name="c", subcore_axis_name="s")

    @functools.partial(
        pl.kernel,
        mesh=mesh,
        out_type=jax.ShapeDtypeStruct((OUT_W, batch), jnp.float32),
        compiler_params=pltpu.CompilerParams(needs_layout_passes=False),
        scratch_types=[
            pltpu.VMEM((b_per_w,), jnp.int32),
            pltpu.VMEM((b_per_w,), jnp.int32),
            pltpu.VMEM((b_per_w,), jnp.int32),
            pltpu.VMEM((2, _L), jnp.float32),
            pltpu.VMEM((OUT_W, b_per_w), jnp.float32),
            pltpu.SemaphoreType.DMA,
            pltpu.SemaphoreType.DMA,
            pltpu.SemaphoreType.DMA,
            pltpu.SemaphoreType.DMA,
            [pltpu.SemaphoreType.DMA] * _GROUPS,
        ],
    )
    def k(t_hbm, h1_hbm, h2_hbm, tab_hbm, out_hbm, t_v, h1_v, h2_v, tab_v,
          out_v, sem_t, sem_h1, sem_h2, sem_tab, sem_g):
        wid = lax.axis_index("s") * _NC + lax.axis_index("c")
        base = wid * b_per_w
        cp_t = pltpu.async_copy(t_hbm.at[pl.ds(base, b_per_w)], t_v, sem_t)
        cp_h1 = pltpu.async_copy(h1_hbm.at[pl.ds(base, b_per_w)], h1_v, sem_h1)
        cp_h2 = pltpu.async_copy(h2_hbm.at[pl.ds(base, b_per_w)], h2_v, sem_h2)
        cp_tabs = [
            pltpu.async_copy(
                tab_hbm.at[r, :],
                tab_v.at[r // 2, pl.ds((r % 2) * EMB_DIM, EMB_DIM)],
                sem_tab)
            for r in range(NUM_TYPES)
        ]
        cp_t.wait()
        cp_h1.wait()
        cp_h2.wait()
        for cp in cp_tabs:
            cp.wait()

        rows = [tab_v[0, :], tab_v[1, :]]
        tab_b = [[jnp.full((_L,), 0.0, jnp.float32)
                  + rows[(r * EMB_DIM + j) // _L][(r * EMB_DIM + j) % _L]
                  for j in range(EMB_DIM)] for r in range(NUM_TYPES)]
        out_cps = []
        for g in range(_GROUPS):
            @pl.loop(0, n_chunks_g, unroll=1)
            def _chunk(cg, g=g):
                sl = pl.ds((g * n_chunks_g + cg) * _L, _L)
                t = t_v[sl]
                m0 = (t & 1) == 1
                m1 = t >= 2
                for j in range(EMB_DIM):
                    lo = jnp.where(m0, tab_b[1][j], tab_b[0][j])
                    hi = jnp.where(m0, tab_b[3][j], tab_b[2][j])
                    out_v[j, sl] = jnp.where(m1, hi, lo)
                fx1, fy1, v1 = _hex_features(h1_v[sl])
                fx2, fy2, v2 = _hex_features(h2_v[sl])
                out_v[EMB_DIM, sl] = fx1
                out_v[EMB_DIM + 1, sl] = fy1
                out_v[EMB_DIM + 2, sl] = v1
                out_v[EMB_DIM + 3, sl] = fx2
                out_v[EMB_DIM + 4, sl] = fy2
                out_v[EMB_DIM + 5, sl] = v2
            g_off = g * g_cols
            out_cps.append(pltpu.async_copy(
                out_v.at[:, pl.ds(g_off, g_cols)],
                out_hbm.at[:, pl.ds(base + g_off, g_cols)],
                sem_g[g]))
        for cp in out_cps:
            cp.wait()

    return k


def kernel(type_idx, hex1, hex2, type_emb):
    batch = type_idx.shape[0]
    k = _make_kernel(batch)
    out_t = k(
        type_idx.astype(jnp.int32),
        hex1.astype(jnp.int32),
        hex2.astype(jnp.int32),
        type_emb.astype(jnp.float32),
    )
    return out_t.T
